# Initial kernel scaffold; baseline (speedup 1.0000x reference)
import jax
import jax.numpy as jnp
from jax.experimental import pallas as pl


def kernel(adj, feat, gcb0_w, gcb0_b, gcb1_w, gcb1_b, gcb2_w, gcb2_b, featgc_w, featgc_b, poolgc_w, poolgc_b, gca0_w, gca0_b, gca1_w, gca1_b, gca2_w, gca2_b, pred_w, pred_b):
    raise NotImplementedError("write your pallas kernel here")


# speedup vs baseline: 1.1329x; 1.1329x over previous
import functools

import jax
import jax.numpy as jnp
from jax import lax
from jax.experimental import pallas as pl
from jax.experimental.pallas import tpu as pltpu

_F32 = jnp.float32


def _gram_kernel(w_ref, b_ref, m_ref, aux_ref):
    w = w_ref[...]
    b = b_ref[...]
    m_ref[...] = lax.dot_general(w, w, (((1,), (1,)), ((), ())),
                                 preferred_element_type=_F32)
    bw = lax.dot_general(b, w, (((1,), (1,)), ((), ())),
                         preferred_element_type=_F32)
    bb = jnp.sum(b * b)
    r = lax.broadcasted_iota(jnp.int32, bw.shape, 0)
    c = lax.broadcasted_iota(jnp.int32, bw.shape, 1)
    aux_ref[...] = bw + jnp.where((r == 1) & (c == 0), bb, 0.0)


def _l2norm(z):
    ssq = jnp.sum(z * z, axis=-1, keepdims=True)
    return z * lax.rsqrt(jnp.maximum(ssq, 1e-24))


def _fused_kernel(adj_ref, feat_ref, w256_ref, w128_ref, b_ref, m_ref, aux_ref,
                  wsel_ref, bsel_ref, ypred_ref, readout_ref,
                  *, hidden, emb, K, L):
    adj = adj_ref[0]
    feat = feat_ref[0]

    deg = jnp.sum(adj, axis=1, keepdims=True)
    adjn = adj / jnp.maximum(deg, 1.0)

    def sage(h, i, dout, relu):
        neigh = jnp.dot(adjn, h, preferred_element_type=_F32)
        cat = jnp.concatenate([h, neigh], axis=-1)
        z = (jnp.dot(cat, w256_ref[:, i * hidden:i * hidden + dout],
                     preferred_element_type=_F32) + b_ref[i:i + 1, 0:dout])
        z = _l2norm(z)
        return jnp.maximum(z, 0.0) if relu else z

    h = sage(feat, 0, hidden, True)
    h = sage(h, 1, hidden, True)
    h = sage(h, 2, emb, False)

    neigh = jnp.dot(adjn, h, preferred_element_type=_F32)
    cat = jnp.concatenate([h, neigh], axis=-1)
    zf = (jnp.dot(cat, w256_ref[:, 3 * hidden:4 * hidden],
                  preferred_element_type=_F32) + b_ref[3:4, 0:hidden])
    zf = jnp.maximum(_l2norm(zf), 0.0)

    y = jnp.dot(cat, m_ref[...], preferred_element_type=_F32)
    t1 = jnp.sum(y * cat, axis=-1, keepdims=True)
    t2 = jnp.sum(cat * aux_ref[0:1, :], axis=-1, keepdims=True)
    ssq = t1 + 2.0 * t2 + aux_ref[1:2, 0:1]
    scale = lax.rsqrt(jnp.maximum(ssq, 1e-24))

    wsel = wsel_ref[0]
    logits = (lax.dot_general(cat, wsel, (((1,), (1,)), ((), ())),
                              preferred_element_type=_F32) + bsel_ref[0])
    zp = jnp.maximum(logits * scale, 0.0)
    mx = jnp.max(zp, axis=-1, keepdims=True)
    e = jnp.exp(zp - mx)
    s = e / jnp.sum(e, axis=-1, keepdims=True)

    hpool = lax.dot_general(s, zf, (((0,), (0,)), ((), ())),
                            preferred_element_type=_F32)
    tmp = jnp.dot(adj, s, preferred_element_type=_F32)
    adjp = lax.dot_general(s, tmp, (((0,), (0,)), ((), ())),
                           preferred_element_type=_F32)

    def bsage(x, i, dout):
        hn = jnp.dot(adjp, x, preferred_element_type=_F32)
        z = (jnp.dot(hn, w128_ref[:, (i - 4) * hidden:(i - 4) * hidden + dout],
                     preferred_element_type=_F32) + b_ref[i:i + 1, 0:dout])
        return jnp.maximum(_l2norm(z), 0.0)

    x = bsage(hpool, 4, hidden)
    x = bsage(x, 5, hidden)
    x = bsage(x, 6, emb)

    readout_ref[0] = x

    KL = K * L
    z = jnp.dot(x, w128_ref[:, 3 * hidden:3 * hidden + KL],
                preferred_element_type=_F32)
    zr = lax.broadcasted_iota(jnp.int32, (K, KL), 0)
    zc = lax.broadcasted_iota(jnp.int32, (K, KL), 1) // L
    z = jnp.where(zr == zc, z, 0.0)
    rr = lax.broadcasted_iota(jnp.int32, (KL, L), 0) % L
    rc = lax.broadcasted_iota(jnp.int32, (KL, L), 1)
    rsel = (rr == rc).astype(_F32)
    d = jnp.dot(z, rsel, preferred_element_type=_F32)
    ones = jnp.ones((K, 1), _F32)
    ypred_ref[0] = (lax.dot_general(ones, d, (((0,), (0,)), ((), ())),
                                    preferred_element_type=_F32)
                    + b_ref[7:8, 0:L])


def kernel(adj, feat, gcb0_w, gcb0_b, gcb1_w, gcb1_b, gcb2_w, gcb2_b,
           featgc_w, featgc_b, poolgc_w, poolgc_b,
           gca0_w, gca0_b, gca1_w, gca1_b, gca2_w, gca2_b, pred_w, pred_b):
    B, N, Din = feat.shape
    hidden = gcb0_w.shape[1]
    emb = gcb2_w.shape[1]
    BK = poolgc_w.shape[1]
    K = BK // B
    L = pred_w.shape[1]
    E2 = poolgc_w.shape[0]

    bp = jnp.zeros((8, BK), _F32).at[0].set(poolgc_b[0])
    m_mat, aux = pl.pallas_call(
        _gram_kernel,
        out_shape=(jax.ShapeDtypeStruct((E2, E2), _F32),
                   jax.ShapeDtypeStruct((8, E2), _F32)),
    )(poolgc_w, bp)

    w256 = jnp.concatenate([gcb0_w, gcb1_w, gcb2_w, featgc_w], axis=1)
    w_cat = pred_w.reshape(K, emb, L).transpose(1, 0, 2).reshape(emb, K * L)
    w_cat_p = jnp.zeros((emb, hidden), _F32).at[:, :K * L].set(w_cat)
    w128 = jnp.concatenate([gca0_w, gca1_w, gca2_w, w_cat_p], axis=1)

    b_all = jnp.zeros((8, max(hidden, 128)), _F32)
    for i, b in enumerate([gcb0_b, gcb1_b, gcb2_b, featgc_b,
                           gca0_b, gca1_b, gca2_b, pred_b]):
        b_all = b_all.at[i, :b.shape[1]].set(b[0])

    wsel = poolgc_w.T.reshape(B, K, E2)
    bsel = poolgc_b.reshape(B, 1, K)

    kern = functools.partial(_fused_kernel, hidden=hidden, emb=emb, K=K, L=L)

    in_specs = [
        pl.BlockSpec((1, N, N), lambda i: (i, 0, 0)),
        pl.BlockSpec((1, N, Din), lambda i: (i, 0, 0)),
        pl.BlockSpec(w256.shape, lambda i: (0, 0)),
        pl.BlockSpec(w128.shape, lambda i: (0, 0)),
        pl.BlockSpec(b_all.shape, lambda i: (0, 0)),
        pl.BlockSpec((E2, E2), lambda i: (0, 0)),
        pl.BlockSpec((8, E2), lambda i: (0, 0)),
        pl.BlockSpec((1, K, E2), lambda i: (i, 0, 0)),
        pl.BlockSpec((1, 1, K), lambda i: (i, 0, 0)),
    ]
    out_specs = (
        pl.BlockSpec((1, 1, L), lambda i: (i, 0, 0)),
        pl.BlockSpec((1, K, emb), lambda i: (i, 0, 0)),
    )

    ypred, readout = pl.pallas_call(
        kern,
        out_shape=(jax.ShapeDtypeStruct((B, 1, L), _F32),
                   jax.ShapeDtypeStruct((B, K, emb), _F32)),
        grid=(B,),
        in_specs=in_specs,
        out_specs=out_specs,
        compiler_params=pltpu.CompilerParams(dimension_semantics=("parallel",)),
    )(adj, feat, w256, w128, b_all, m_mat, aux, wsel, bsel)

    return ypred.reshape(B, L), readout.reshape(B, K * emb)


# speedup vs baseline: 5.4274x; 4.7907x over previous
import functools

import jax
import jax.numpy as jnp
from jax import lax
from jax.experimental import pallas as pl
from jax.experimental.pallas import tpu as pltpu

_F32 = jnp.float32


def _gram_kernel(w_ref, b_ref, m_ref, aux_ref):
    w = w_ref[...]
    b = b_ref[...]
    m_ref[...] = lax.dot_general(w, w, (((1,), (1,)), ((), ())),
                                 preferred_element_type=_F32)
    bw = lax.dot_general(b, w, (((1,), (1,)), ((), ())),
                         preferred_element_type=_F32)
    bb = jnp.sum(b * b)
    r = lax.broadcasted_iota(jnp.int32, bw.shape, 0)
    c = lax.broadcasted_iota(jnp.int32, bw.shape, 1)
    aux_ref[...] = bw + jnp.where((r == 1) & (c == 0), bb, 0.0)


def _l2norm(z):
    ssq = jnp.sum(z * z, axis=-1, keepdims=True)
    return z * lax.rsqrt(jnp.maximum(ssq, 1e-24))


def _fused_kernel(adj_ref, feat_ref, w256_ref, w128_ref, b_ref, m_ref, aux_ref,
                  wsel_ref, bsel_ref, ypred_ref, readout_ref,
                  *, N, hidden, emb, K, L, G):
    GN = G * N
    GK = G * K
    adj = adj_ref[0]
    feat = feat_ref[...].reshape(GN, -1)

    deg = jnp.sum(adj, axis=2, keepdims=True)
    recip = (1.0 / jnp.maximum(deg, 1.0)).reshape(GN, 1)

    def agg(h):
        parts = [jnp.dot(adj[g], h[g * N:(g + 1) * N],
                         preferred_element_type=_F32) for g in range(G)]
        return jnp.concatenate(parts, axis=0) * recip

    def sage(h, i, dout, relu):
        cat = jnp.concatenate([h, agg(h)], axis=-1)
        z = (jnp.dot(cat, w256_ref[:, i * hidden:i * hidden + dout],
                     preferred_element_type=_F32) + b_ref[i:i + 1, 0:dout])
        z = _l2norm(z)
        return jnp.maximum(z, 0.0) if relu else z

    h = sage(feat, 0, hidden, True)
    h = sage(h, 1, hidden, True)
    h = sage(h, 2, emb, False)

    cat = jnp.concatenate([h, agg(h)], axis=-1)
    zf = (jnp.dot(cat, w256_ref[:, 3 * hidden:4 * hidden],
                  preferred_element_type=_F32) + b_ref[3:4, 0:hidden])
    zf = jnp.maximum(_l2norm(zf), 0.0)

    y = jnp.dot(cat, m_ref[...], preferred_element_type=_F32)
    t1 = jnp.sum(y * cat, axis=-1, keepdims=True)
    t2 = jnp.sum(cat * aux_ref[0:1, :], axis=-1, keepdims=True)
    ssq = t1 + 2.0 * t2 + aux_ref[1:2, 0:1]
    scale = lax.rsqrt(jnp.maximum(ssq, 1e-24))

    logits = (jnp.dot(cat, wsel_ref[0], preferred_element_type=_F32)
              + bsel_ref[0])
    zp = jnp.maximum(logits * scale, 0.0)
    rg = lax.broadcasted_iota(jnp.int32, (GN, GK), 0) // N
    cg = lax.broadcasted_iota(jnp.int32, (GN, GK), 1) // K
    zp = jnp.where(rg == cg, zp, -1e30)
    mx = jnp.max(zp, axis=-1, keepdims=True)
    e = jnp.exp(zp - mx)
    s = e / jnp.sum(e, axis=-1, keepdims=True)

    hpool = lax.dot_general(s, zf, (((0,), (0,)), ((), ())),
                            preferred_element_type=_F32)
    tmp = jnp.concatenate(
        [jnp.dot(adj[g], s[g * N:(g + 1) * N], preferred_element_type=_F32)
         for g in range(G)], axis=0)
    adjp = lax.dot_general(s, tmp, (((0,), (0,)), ((), ())),
                           preferred_element_type=_F32)

    def bsage(x, i, dout):
        hn = jnp.dot(adjp, x, preferred_element_type=_F32)
        z = (jnp.dot(hn, w128_ref[:, (i - 4) * hidden:(i - 4) * hidden + dout],
                     preferred_element_type=_F32) + b_ref[i:i + 1, 0:dout])
        return jnp.maximum(_l2norm(z), 0.0)

    x = bsage(hpool, 4, hidden)
    x = bsage(x, 5, hidden)
    x = bsage(x, 6, emb)

    readout_ref[0] = x

    KL = K * L
    z = jnp.dot(x, w128_ref[:, 3 * hidden:3 * hidden + KL],
                preferred_element_type=_F32)
    zr = lax.broadcasted_iota(jnp.int32, (GK, KL), 0) % K
    zc = lax.broadcasted_iota(jnp.int32, (GK, KL), 1) // L
    z = jnp.where(zr == zc, z, 0.0)
    rr = lax.broadcasted_iota(jnp.int32, (KL, L), 0) % L
    rc = lax.broadcasted_iota(jnp.int32, (KL, L), 1)
    rsel = (rr == rc).astype(_F32)
    d = jnp.dot(z, rsel, preferred_element_type=_F32)
    gr = lax.broadcasted_iota(jnp.int32, (GK, G), 0) // K
    gc = lax.broadcasted_iota(jnp.int32, (GK, G), 1)
    gsel = (gr == gc).astype(_F32)
    ypred_ref[0] = (lax.dot_general(gsel, d, (((0,), (0,)), ((), ())),
                                    preferred_element_type=_F32)
                    + b_ref[7:8, 0:L])


def kernel(adj, feat, gcb0_w, gcb0_b, gcb1_w, gcb1_b, gcb2_w, gcb2_b,
           featgc_w, featgc_b, poolgc_w, poolgc_b,
           gca0_w, gca0_b, gca1_w, gca1_b, gca2_w, gca2_b, pred_w, pred_b):
    B, N, Din = feat.shape
    hidden = gcb0_w.shape[1]
    emb = gcb2_w.shape[1]
    BK = poolgc_w.shape[1]
    K = BK // B
    L = pred_w.shape[1]
    E2 = poolgc_w.shape[0]

    G = 1
    for d in (8, 4, 2):
        if B % d == 0:
            G = d
            break
    NG = B // G
    GK = G * K

    bp = jnp.zeros((8, BK), _F32).at[0].set(poolgc_b[0])
    m_mat, aux = pl.pallas_call(
        _gram_kernel,
        out_shape=(jax.ShapeDtypeStruct((E2, E2), _F32),
                   jax.ShapeDtypeStruct((8, E2), _F32)),
    )(poolgc_w, bp)

    w256 = jnp.concatenate([gcb0_w, gcb1_w, gcb2_w, featgc_w], axis=1)
    w_cat = pred_w.reshape(K, emb, L).transpose(1, 0, 2).reshape(emb, K * L)
    w_cat_p = jnp.zeros((emb, hidden), _F32).at[:, :K * L].set(w_cat)
    w128 = jnp.concatenate([gca0_w, gca1_w, gca2_w, w_cat_p], axis=1)

    b_all = jnp.zeros((8, max(hidden, 128)), _F32)
    for i, b in enumerate([gcb0_b, gcb1_b, gcb2_b, featgc_b,
                           gca0_b, gca1_b, gca2_b, pred_b]):
        b_all = b_all.at[i, :b.shape[1]].set(b[0])

    wsel = poolgc_w.reshape(E2, NG, GK).transpose(1, 0, 2)
    bsel = poolgc_b.reshape(NG, 1, GK)

    adj4 = adj.reshape(NG, G, N, N)
    feat4 = feat.reshape(NG, G * N, Din)

    kern = functools.partial(_fused_kernel, N=N, hidden=hidden, emb=emb,
                             K=K, L=L, G=G)

    in_specs = [
        pl.BlockSpec((1, G, N, N), lambda i: (i, 0, 0, 0)),
        pl.BlockSpec((1, G * N, Din), lambda i: (i, 0, 0)),
        pl.BlockSpec(w256.shape, lambda i: (0, 0)),
        pl.BlockSpec(w128.shape, lambda i: (0, 0)),
        pl.BlockSpec(b_all.shape, lambda i: (0, 0)),
        pl.BlockSpec((E2, E2), lambda i: (0, 0)),
        pl.BlockSpec((8, E2), lambda i: (0, 0)),
        pl.BlockSpec((1, E2, GK), lambda i: (i, 0, 0)),
        pl.BlockSpec((1, 1, GK), lambda i: (i, 0, 0)),
    ]
    out_specs = (
        pl.BlockSpec((1, G, L), lambda i: (i, 0, 0)),
        pl.BlockSpec((1, GK, emb), lambda i: (i, 0, 0)),
    )

    ypred, readout = pl.pallas_call(
        kern,
        out_shape=(jax.ShapeDtypeStruct((NG, G, L), _F32),
                   jax.ShapeDtypeStruct((NG, GK, emb), _F32)),
        grid=(NG,),
        in_specs=in_specs,
        out_specs=out_specs,
        compiler_params=pltpu.CompilerParams(dimension_semantics=("parallel",)),
    )(adj4, feat4, w256, w128, b_all, m_mat, aux, wsel, bsel)

    return ypred.reshape(B, L), readout.reshape(B, K * emb)
